# baseline (speedup 1.0000x reference)
import jax
import jax.numpy as jnp
from jax import lax
from jax.experimental import pallas as pl
from jax.experimental.pallas import tpu as pltpu
from jax.experimental.pallas import tpu_sc as plsc

NUM_ENTITIES = 1000000
NUM_RELATIONS = 1000
EMBED_DIM = 64
BATCH = 16384

NC, NS, L = 2, 16, 16
NW = NC * NS
BPW = BATCH // NW
NG = BPW // L
NB = 4
EPS = 1e-12
DK = EMBED_DIM // L


def _newton_rsqrt(t):
    ti = plsc.bitcast(t, jnp.int32)
    yi = jnp.int32(0x5F3759DF) - lax.shift_right_logical(ti, 1)
    y = plsc.bitcast(yi, jnp.float32)
    th = t * 0.5
    for _ in range(3):
        y = y * (1.5 - th * y * y)
    return y


def _tec_body(src_hbm, rel_hbm, ent_hbm, reltab_hbm, out_hbm,
              src_v, rel_v, sign_v, ering, rring, o_v, sems, osems):
    cid = lax.axis_index("c")
    sid = lax.axis_index("s")
    wid = sid * NC + cid
    base = wid * BPW

    pltpu.sync_copy(src_hbm.at[pl.ds(base, BPW)], src_v)
    pltpu.sync_copy(rel_hbm.at[pl.ds(base, BPW)], rel_v)

    for i in range(NG):
        sl = pl.ds(i * L, L)
        rv = rel_v[sl]
        ge = rv >= NUM_RELATIONS
        rel_v[sl] = jnp.where(ge, rv - NUM_RELATIONS, rv)
        sign_v[sl] = jnp.where(ge, jnp.float32(-1.0), jnp.float32(1.0))

    def issue(g, buf):
        sv = src_v[pl.ds(g * L, L)]
        rm = rel_v[pl.ds(g * L, L)]
        for j in range(L):
            pltpu.make_async_copy(
                ent_hbm.at[sv[j]], ering.at[buf, j], sems.at[buf, 0]
            ).start()
            pltpu.make_async_copy(
                reltab_hbm.at[rm[j]], rring.at[buf, j], sems.at[buf, 1]
            ).start()

    def drain(buf):
        pltpu.make_async_copy(
            ent_hbm.at[pl.ds(0, L)], ering.at[buf], sems.at[buf, 0]
        ).wait()
        pltpu.make_async_copy(
            ent_hbm.at[pl.ds(0, L)], rring.at[buf], sems.at[buf, 1]
        ).wait()

    def compute(g, buf, obuf):
        sgn_vec = sign_v[pl.ds(g * L, L)]
        for j in range(L):
            e = [ering[buf, j, pl.ds(k * L, L)] for k in range(DK)]
            r_ = [rring[buf, j, pl.ds(k * L, L)] for k in range(DK)]
            sq_e = e[0] * e[0]
            sq_r = r_[0] * r_[0]
            for k in range(1, DK):
                sq_e = sq_e + e[k] * e[k]
                sq_r = sq_r + r_[k] * r_[k]
            te = jnp.maximum(jnp.sum(sq_e), jnp.float32(EPS))
            tr = jnp.maximum(jnp.sum(sq_r), jnp.float32(EPS))
            inv_e = _newton_rsqrt(jnp.full((L,), te, jnp.float32))
            inv_r = _newton_rsqrt(jnp.full((L,), tr, jnp.float32))
            inv_rs = inv_r * sgn_vec[j]
            for k in range(DK):
                o_v[obuf, j, pl.ds(k * L, L)] = (
                    e[k] * inv_e + r_[k] * inv_rs)

    def owait(obuf):
        pltpu.make_async_copy(
            ent_hbm.at[pl.ds(0, L)], o_v.at[obuf], osems.at[obuf]
        ).wait()

    for p in range(NB - 1):
        issue(p, p)

    def body(g, _):
        buf = g & (NB - 1)
        obuf = g & 1

        @pl.when(g + NB - 1 < NG)
        def _():
            issue(g + NB - 1, (g + NB - 1) & (NB - 1))

        drain(buf)

        @pl.when(g >= 2)
        def _():
            owait(obuf)

        compute(g, buf, obuf)
        pltpu.make_async_copy(
            o_v.at[obuf], out_hbm.at[pl.ds(base + g * L, L)], osems.at[obuf]
        ).start()
        return _

    lax.fori_loop(0, NG, body, None)
    owait(0)
    owait(1)


@jax.jit
def kernel(source, relations, entity_embeddings, relation_embeddings):
    mesh = plsc.VectorSubcoreMesh(
        core_axis_name="c", subcore_axis_name="s", num_cores=NC, num_subcores=NS
    )
    run = pl.kernel(
        _tec_body,
        out_type=jax.ShapeDtypeStruct((BATCH, EMBED_DIM), jnp.float32),
        mesh=mesh,
        compiler_params=pltpu.CompilerParams(needs_layout_passes=False),
        scratch_types=[
            pltpu.VMEM((BPW,), jnp.int32),
            pltpu.VMEM((BPW,), jnp.int32),
            pltpu.VMEM((BPW,), jnp.float32),
            pltpu.VMEM((NB, L, EMBED_DIM), jnp.float32),
            pltpu.VMEM((NB, L, EMBED_DIM), jnp.float32),
            pltpu.VMEM((2, L, EMBED_DIM), jnp.float32),
            pltpu.SemaphoreType.DMA((NB, 2)),
            pltpu.SemaphoreType.DMA((2,)),
        ],
    )
    return run(
        source.astype(jnp.int32),
        relations.astype(jnp.int32),
        entity_embeddings,
        relation_embeddings,
    )
